# baseline (speedup 1.0000x reference)
import functools

import jax
import jax.numpy as jnp
from jax import lax
from jax.experimental import pallas as pl
from jax.experimental.pallas import tpu as pltpu
from jax.experimental.pallas import tpu_sc as plsc

N = 10000
NP = 10240
E = 320000
H = 128
NC = 2
NS = 16
NW = NC * NS
EPW = E // NW
C = 64
NCHUNK = 157
RPT = NP // NS
DW = 16

_HIGH = lax.Precision.HIGHEST

_mesh = plsc.VectorSubcoreMesh(
    core_axis_name="c", subcore_axis_name="s", num_cores=NC, num_subcores=NS)


@functools.partial(
    pl.kernel,
    mesh=_mesh,
    out_type=jax.ShapeDtypeStruct((2 * NP, H), jnp.float32),
    scratch_types=[
        pltpu.VMEM((NCHUNK, C), jnp.int32),
        pltpu.VMEM((NCHUNK, C), jnp.int32),
        pltpu.VMEM((C, H), jnp.float32),
        pltpu.VMEM_SHARED((NP, H), jnp.float32),
        pltpu.SemaphoreType.DMA,
    ],
)
def _segsum_sc(p_hbm, src3, dst3, zeros_hbm, out_hbm,
               sslab, dslab, rows, acc, sem):
    cid = lax.axis_index("c")
    sid = lax.axis_index("s")
    wid = cid * NS + sid
    pltpu.sync_copy(zeros_hbm.at[pl.ds(sid * RPT, RPT), :],
                    acc.at[pl.ds(sid * RPT, RPT), :])
    pltpu.sync_copy(src3.at[wid], sslab)
    pltpu.sync_copy(dst3.at[wid], dslab)
    plsc.subcore_barrier()

    def body(i, carry):
        pltpu.async_copy(p_hbm.at[sslab.at[i]], rows, sem).wait()
        pltpu.sync_copy(rows, acc.at[dslab.at[i]], add=True)
        return carry

    lax.fori_loop(0, NCHUNK, body, 0)
    plsc.subcore_barrier()
    pltpu.sync_copy(acc.at[pl.ds(sid * RPT, RPT), :],
                    out_hbm.at[pl.ds(cid * NP + sid * RPT, RPT), :])


@functools.partial(
    pl.kernel,
    mesh=_mesh,
    out_type=jax.ShapeDtypeStruct((2 * NP, DW), jnp.float32),
    scratch_types=[
        pltpu.VMEM((NCHUNK, C), jnp.int32),
        pltpu.VMEM((C, DW), jnp.float32),
        pltpu.VMEM_SHARED((NP, DW), jnp.float32),
    ],
)
def _deg_sc(dst3, zeros_hbm, out_hbm, dslab, ones_v, deg_sh):
    cid = lax.axis_index("c")
    sid = lax.axis_index("s")
    wid = cid * NS + sid
    pltpu.sync_copy(zeros_hbm.at[pl.ds(sid * RPT, RPT), :],
                    deg_sh.at[pl.ds(sid * RPT, RPT), :])
    pltpu.sync_copy(dst3.at[wid], dslab)
    for j in range(C):
        ones_v[j, :] = jnp.full((DW,), 1.0, jnp.float32)
    plsc.subcore_barrier()

    def body(k, carry):
        pltpu.sync_copy(ones_v, deg_sh.at[dslab.at[k]], add=True)
        return carry

    lax.fori_loop(0, NCHUNK, body, 0)
    plsc.subcore_barrier()
    pltpu.sync_copy(deg_sh.at[pl.ds(sid * RPT, RPT), :],
                    out_hbm.at[pl.ds(cid * NP + sid * RPT, RPT), :])


BN = 2000


def _emb_pq_body(x_ref, we_ref, be_ref, wl_ref, wr_ref, p_ref, q_ref):
    hh = jax.nn.relu(
        jnp.dot(x_ref[...], we_ref[...], precision=_HIGH,
                preferred_element_type=jnp.float32) + be_ref[...])
    p_ref[...] = jnp.dot(hh, wl_ref[...], precision=_HIGH,
                         preferred_element_type=jnp.float32)
    q_ref[...] = jnp.dot(hh, wr_ref[...], precision=_HIGH,
                         preferred_element_type=jnp.float32)


def _deg_body(d0_ref, d1_ref, out_ref):
    d = jnp.maximum(d0_ref[:, 0:1] + d1_ref[:, 0:1], 1.0)
    out_ref[...] = jnp.broadcast_to(d, (BN, H))


def _comb_pq_body(a0_ref, a1_ref, dg_ref, q_ref, b_ref, wl_ref, wr_ref,
                  p_ref, qn_ref):
    hh = jax.nn.relu((a0_ref[...] + a1_ref[...]) / dg_ref[...]
                     + b_ref[...] + q_ref[...])
    p_ref[...] = jnp.dot(hh, wl_ref[...], precision=_HIGH,
                         preferred_element_type=jnp.float32)
    qn_ref[...] = jnp.dot(hh, wr_ref[...], precision=_HIGH,
                          preferred_element_type=jnp.float32)


def _comb_body(a0_ref, a1_ref, dg_ref, q_ref, b_ref, h_ref):
    h_ref[...] = jax.nn.relu((a0_ref[...] + a1_ref[...]) / dg_ref[...]
                             + b_ref[...] + q_ref[...])


def _row_spec():
    return pl.BlockSpec((BN, H), lambda i: (i, 0))


def _w_spec():
    return pl.BlockSpec((H, H), lambda i: (0, 0))


def _b_spec():
    return pl.BlockSpec((1, H), lambda i: (0, 0))


def _f32(shape):
    return jax.ShapeDtypeStruct(shape, jnp.float32)


def kernel(x, edge_index, W_emb, b_emb, Wl0, bl0, Wr0, Wl1, bl1, Wr1,
           Wl2, bl2, Wr2, Wl3, bl3, Wr3):
    grid = (N // BN,)
    pad = NCHUNK * C - EPW
    src3 = jnp.pad(edge_index[0].reshape(NW, EPW),
                   ((0, 0), (0, pad))).reshape(NW, NCHUNK, C)
    dst3 = jnp.pad(edge_index[1].reshape(NW, EPW), ((0, 0), (0, pad)),
                   constant_values=NP - 1).reshape(NW, NCHUNK, C)
    zeros = jnp.zeros((NP, H), jnp.float32)
    zeros16 = jnp.zeros((NP, DW), jnp.float32)

    ones = jnp.ones((N, H), jnp.float32)
    dacc = _segsum_sc(ones, src3, dst3, zeros)
    degc = pl.pallas_call(
        _deg_body, grid=grid,
        in_specs=[pl.BlockSpec((BN, H), lambda i: (i, 0)),
                  pl.BlockSpec((BN, H), lambda i: (i, 0))],
        out_specs=_row_spec(), out_shape=_f32((N, H)),
    )(dacc[:N], dacc[NP:NP + N])

    p, q = pl.pallas_call(
        _emb_pq_body, grid=grid,
        in_specs=[_row_spec(), _w_spec(), _b_spec(), _w_spec(), _w_spec()],
        out_specs=[_row_spec(), _row_spec()],
        out_shape=[_f32((N, H)), _f32((N, H))],
    )(x, W_emb, b_emb.reshape(1, H), Wl0, Wr0)
    p, dacc = lax.optimization_barrier((p, dacc))

    layer_b = [bl0, bl1, bl2, bl3]
    next_w = [(Wl1, Wr1), (Wl2, Wr2), (Wl3, Wr3), None]
    h = None
    for li in range(4):
        acc = _segsum_sc(p, src3, dst3, zeros)
        a0, a1 = acc[:N], acc[NP:NP + N]
        bl = layer_b[li].reshape(1, H)
        if next_w[li] is not None:
            wl_n, wr_n = next_w[li]
            p, q = pl.pallas_call(
                _comb_pq_body, grid=grid,
                in_specs=[_row_spec(), _row_spec(), _row_spec(), _row_spec(),
                          _b_spec(), _w_spec(), _w_spec()],
                out_specs=[_row_spec(), _row_spec()],
                out_shape=[_f32((N, H)), _f32((N, H))],
            )(a0, a1, degc, q, bl, wl_n, wr_n)
        else:
            h = pl.pallas_call(
                _comb_body, grid=grid,
                in_specs=[_row_spec(), _row_spec(), _row_spec(), _row_spec(),
                          _b_spec()],
                out_specs=_row_spec(), out_shape=_f32((N, H)),
            )(a0, a1, degc, q, bl)
    return h


# speedup vs baseline: 1.2976x; 1.2976x over previous
import functools

import jax
import jax.numpy as jnp
from jax import lax
from jax.experimental import pallas as pl
from jax.experimental.pallas import tpu as pltpu
from jax.experimental.pallas import tpu_sc as plsc

N = 10000
NP = 10240
E = 320000
H = 128
NC = 2
NS = 16
NW = NC * NS
EPW = E // NW
C = 80
NCHUNK = 125
RPT = NP // NS
DW = 16

_HIGH = lax.Precision.HIGHEST

_mesh = plsc.VectorSubcoreMesh(
    core_axis_name="c", subcore_axis_name="s", num_cores=NC, num_subcores=NS)


@functools.partial(
    pl.kernel,
    mesh=_mesh,
    out_type=jax.ShapeDtypeStruct((2 * NP, H), jnp.float32),
    scratch_types=[
        pltpu.VMEM((NCHUNK, C), jnp.int32),
        pltpu.VMEM((NCHUNK, C), jnp.int32),
        pltpu.VMEM((C, H), jnp.float32),
        pltpu.VMEM_SHARED((NP, H), jnp.float32),
        pltpu.SemaphoreType.DMA,
    ],
)
def _segsum_sc(p_hbm, src3, dst3, zeros_hbm, out_hbm,
               sslab, dslab, rows, acc, sem):
    cid = lax.axis_index("c")
    sid = lax.axis_index("s")
    wid = cid * NS + sid
    pltpu.sync_copy(zeros_hbm.at[pl.ds(sid * RPT, RPT), :],
                    acc.at[pl.ds(sid * RPT, RPT), :])
    pltpu.sync_copy(src3.at[wid], sslab)
    pltpu.sync_copy(dst3.at[wid], dslab)
    plsc.subcore_barrier()

    def body(i, carry):
        pltpu.async_copy(p_hbm.at[sslab.at[i]], rows, sem).wait()
        pltpu.sync_copy(rows, acc.at[dslab.at[i]], add=True)
        return carry

    lax.fori_loop(0, NCHUNK, body, 0)
    plsc.subcore_barrier()
    pltpu.sync_copy(acc.at[pl.ds(sid * RPT, RPT), :],
                    out_hbm.at[pl.ds(cid * NP + sid * RPT, RPT), :])


@functools.partial(
    pl.kernel,
    mesh=_mesh,
    out_type=jax.ShapeDtypeStruct((2 * NP, DW), jnp.float32),
    scratch_types=[
        pltpu.VMEM((NCHUNK, C), jnp.int32),
        pltpu.VMEM((C, DW), jnp.float32),
        pltpu.VMEM_SHARED((NP, DW), jnp.float32),
    ],
)
def _deg_sc(dst3, zeros_hbm, out_hbm, dslab, ones_v, deg_sh):
    cid = lax.axis_index("c")
    sid = lax.axis_index("s")
    wid = cid * NS + sid
    pltpu.sync_copy(zeros_hbm.at[pl.ds(sid * RPT, RPT), :],
                    deg_sh.at[pl.ds(sid * RPT, RPT), :])
    pltpu.sync_copy(dst3.at[wid], dslab)
    for j in range(C):
        ones_v[j, :] = jnp.full((DW,), 1.0, jnp.float32)
    plsc.subcore_barrier()

    def body(k, carry):
        pltpu.sync_copy(ones_v, deg_sh.at[dslab.at[k]], add=True)
        return carry

    lax.fori_loop(0, NCHUNK, body, 0)
    plsc.subcore_barrier()
    pltpu.sync_copy(deg_sh.at[pl.ds(sid * RPT, RPT), :],
                    out_hbm.at[pl.ds(cid * NP + sid * RPT, RPT), :])


BN = 2000


def _emb_pq_body(x_ref, we_ref, be_ref, wl_ref, wr_ref, p_ref, q_ref):
    hh = jax.nn.relu(
        jnp.dot(x_ref[...], we_ref[...], precision=_HIGH,
                preferred_element_type=jnp.float32) + be_ref[...])
    p_ref[...] = jnp.dot(hh, wl_ref[...], precision=_HIGH,
                         preferred_element_type=jnp.float32)
    q_ref[...] = jnp.dot(hh, wr_ref[...], precision=_HIGH,
                         preferred_element_type=jnp.float32)


def _deg_body(d0_ref, d1_ref, out_ref):
    d = jnp.maximum(d0_ref[:, 0:1] + d1_ref[:, 0:1], 1.0)
    out_ref[...] = jnp.broadcast_to(d, (BN, H))


def _comb_pq_body(a0_ref, a1_ref, dg_ref, q_ref, b_ref, wl_ref, wr_ref,
                  p_ref, qn_ref):
    hh = jax.nn.relu((a0_ref[...] + a1_ref[...]) / dg_ref[...]
                     + b_ref[...] + q_ref[...])
    p_ref[...] = jnp.dot(hh, wl_ref[...], precision=_HIGH,
                         preferred_element_type=jnp.float32)
    qn_ref[...] = jnp.dot(hh, wr_ref[...], precision=_HIGH,
                          preferred_element_type=jnp.float32)


def _comb_body(a0_ref, a1_ref, dg_ref, q_ref, b_ref, h_ref):
    h_ref[...] = jax.nn.relu((a0_ref[...] + a1_ref[...]) / dg_ref[...]
                             + b_ref[...] + q_ref[...])


def _row_spec():
    return pl.BlockSpec((BN, H), lambda i: (i, 0))


def _w_spec():
    return pl.BlockSpec((H, H), lambda i: (0, 0))


def _b_spec():
    return pl.BlockSpec((1, H), lambda i: (0, 0))


def _f32(shape):
    return jax.ShapeDtypeStruct(shape, jnp.float32)


def kernel(x, edge_index, W_emb, b_emb, Wl0, bl0, Wr0, Wl1, bl1, Wr1,
           Wl2, bl2, Wr2, Wl3, bl3, Wr3):
    grid = (N // BN,)
    src3 = edge_index[0].reshape(NW, NCHUNK, C)
    dst3 = edge_index[1].reshape(NW, NCHUNK, C)
    zeros = jnp.zeros((NP, H), jnp.float32)
    zeros16 = jnp.zeros((NP, DW), jnp.float32)

    ones = jnp.ones((N, H), jnp.float32)
    dacc = _segsum_sc(ones, src3, dst3, zeros)
    degc = pl.pallas_call(
        _deg_body, grid=grid,
        in_specs=[pl.BlockSpec((BN, H), lambda i: (i, 0)),
                  pl.BlockSpec((BN, H), lambda i: (i, 0))],
        out_specs=_row_spec(), out_shape=_f32((N, H)),
    )(dacc[:N], dacc[NP:NP + N])

    p, q = pl.pallas_call(
        _emb_pq_body, grid=grid,
        in_specs=[_row_spec(), _w_spec(), _b_spec(), _w_spec(), _w_spec()],
        out_specs=[_row_spec(), _row_spec()],
        out_shape=[_f32((N, H)), _f32((N, H))],
    )(x, W_emb, b_emb.reshape(1, H), Wl0, Wr0)
    p, dacc = lax.optimization_barrier((p, dacc))

    layer_b = [bl0, bl1, bl2, bl3]
    next_w = [(Wl1, Wr1), (Wl2, Wr2), (Wl3, Wr3), None]
    h = None
    for li in range(4):
        acc = _segsum_sc(p, src3, dst3, zeros)
        a0, a1 = acc[:N], acc[NP:NP + N]
        bl = layer_b[li].reshape(1, H)
        if next_w[li] is not None:
            wl_n, wr_n = next_w[li]
            p, q = pl.pallas_call(
                _comb_pq_body, grid=grid,
                in_specs=[_row_spec(), _row_spec(), _row_spec(), _row_spec(),
                          _b_spec(), _w_spec(), _w_spec()],
                out_specs=[_row_spec(), _row_spec()],
                out_shape=[_f32((N, H)), _f32((N, H))],
            )(a0, a1, degc, q, bl, wl_n, wr_n)
        else:
            h = pl.pallas_call(
                _comb_body, grid=grid,
                in_specs=[_row_spec(), _row_spec(), _row_spec(), _row_spec(),
                          _b_spec()],
                out_specs=_row_spec(), out_shape=_f32((N, H)),
            )(a0, a1, degc, q, bl)
    return h


# speedup vs baseline: 1.4763x; 1.1378x over previous
import functools

import jax
import jax.numpy as jnp
from jax import lax
from jax.experimental import pallas as pl
from jax.experimental.pallas import tpu as pltpu
from jax.experimental.pallas import tpu_sc as plsc

N = 10000
NP = 10240
E = 320000
H = 128
NC = 2
NS = 16
NW = NC * NS
EPW = E // NW
C = 80
NCHUNK = 125
RPT = NP // NS
DW = 16

_HIGH = lax.Precision.HIGHEST

_mesh = plsc.VectorSubcoreMesh(
    core_axis_name="c", subcore_axis_name="s", num_cores=NC, num_subcores=NS)


@functools.partial(
    pl.kernel,
    mesh=_mesh,
    out_type=jax.ShapeDtypeStruct((2 * NP, H), jnp.float32),
    scratch_types=[
        pltpu.VMEM((NCHUNK, C), jnp.int32),
        pltpu.VMEM((NCHUNK, C), jnp.int32),
        pltpu.VMEM((C, H), jnp.float32),
        pltpu.VMEM_SHARED((NP, H), jnp.float32),
        pltpu.SemaphoreType.DMA,
    ],
)
def _segsum_sc(p_hbm, src3, dst3, zeros_hbm, out_hbm,
               sslab, dslab, rows, acc, sem):
    cid = lax.axis_index("c")
    sid = lax.axis_index("s")
    wid = cid * NS + sid
    pltpu.sync_copy(zeros_hbm.at[pl.ds(sid * RPT, RPT), :],
                    acc.at[pl.ds(sid * RPT, RPT), :])
    pltpu.sync_copy(src3.at[wid], sslab)
    pltpu.sync_copy(dst3.at[wid], dslab)
    plsc.subcore_barrier()

    def body(i, carry):
        pltpu.async_copy(p_hbm.at[sslab.at[i]], rows, sem).wait()
        pltpu.sync_copy(rows, acc.at[dslab.at[i]], add=True)
        return carry

    lax.fori_loop(0, NCHUNK, body, 0)
    plsc.subcore_barrier()
    pltpu.sync_copy(acc.at[pl.ds(sid * RPT, RPT), :],
                    out_hbm.at[pl.ds(cid * NP + sid * RPT, RPT), :])


@functools.partial(
    pl.kernel,
    mesh=_mesh,
    out_type=jax.ShapeDtypeStruct((2 * NP, DW), jnp.float32),
    scratch_types=[
        pltpu.VMEM((NCHUNK, C), jnp.int32),
        pltpu.VMEM((C, DW), jnp.float32),
        pltpu.VMEM_SHARED((NP, DW), jnp.float32),
    ],
)
def _deg_sc(dst3, zeros_hbm, out_hbm, dslab, ones_v, deg_sh):
    cid = lax.axis_index("c")
    sid = lax.axis_index("s")
    wid = cid * NS + sid
    pltpu.sync_copy(zeros_hbm.at[pl.ds(sid * RPT, RPT), :],
                    deg_sh.at[pl.ds(sid * RPT, RPT), :])
    pltpu.sync_copy(dst3.at[wid], dslab)
    for j in range(C):
        ones_v[j, :] = jnp.full((DW,), 1.0, jnp.float32)
    plsc.subcore_barrier()

    def body(k, carry):
        pltpu.sync_copy(ones_v, deg_sh.at[dslab.at[k]], add=True)
        return carry

    lax.fori_loop(0, NCHUNK, body, 0)
    plsc.subcore_barrier()
    pltpu.sync_copy(deg_sh.at[pl.ds(sid * RPT, RPT), :],
                    out_hbm.at[pl.ds(cid * NP + sid * RPT, RPT), :])


@functools.partial(
    pl.kernel,
    mesh=_mesh,
    out_type=jax.ShapeDtypeStruct((2 * NP, H), jnp.float32),
    scratch_types=[
        pltpu.VMEM((NCHUNK, C), jnp.int32),
        pltpu.VMEM((C, H), jnp.float32),
        pltpu.VMEM_SHARED((NP, H), jnp.float32),
    ],
)
def _deg_sc(dst3, zeros_hbm, out_hbm, dslab, ones_v, deg_sh):
    cid = lax.axis_index("c")
    sid = lax.axis_index("s")
    wid = cid * NS + sid
    pltpu.sync_copy(zeros_hbm.at[pl.ds(sid * RPT, RPT), :],
                    deg_sh.at[pl.ds(sid * RPT, RPT), :])
    pltpu.sync_copy(dst3.at[wid], dslab)

    def fill(j, carry):
        for l in range(H // 16):
            ones_v[j, pl.ds(l * 16, 16)] = jnp.full((16,), 1.0, jnp.float32)
        return carry

    lax.fori_loop(0, C, fill, 0)
    plsc.subcore_barrier()

    def body(k, carry):
        pltpu.sync_copy(ones_v, deg_sh.at[dslab.at[k]], add=True)
        return carry

    lax.fori_loop(0, NCHUNK, body, 0)
    plsc.subcore_barrier()
    pltpu.sync_copy(deg_sh.at[pl.ds(sid * RPT, RPT), :],
                    out_hbm.at[pl.ds(cid * NP + sid * RPT, RPT), :])


BN = 2000


def _emb_pq_body(x_ref, we_ref, be_ref, wl_ref, wr_ref, p_ref, q_ref):
    hh = jax.nn.relu(
        jnp.dot(x_ref[...], we_ref[...], precision=_HIGH,
                preferred_element_type=jnp.float32) + be_ref[...])
    p_ref[...] = jnp.dot(hh, wl_ref[...], precision=_HIGH,
                         preferred_element_type=jnp.float32)
    q_ref[...] = jnp.dot(hh, wr_ref[...], precision=_HIGH,
                         preferred_element_type=jnp.float32)


def _deg_body(d0_ref, d1_ref, out_ref):
    d = jnp.maximum(d0_ref[:, 0:1] + d1_ref[:, 0:1], 1.0)
    out_ref[...] = jnp.broadcast_to(d, (BN, H))


def _comb_pq_body(a0_ref, a1_ref, dg_ref, q_ref, b_ref, wl_ref, wr_ref,
                  p_ref, qn_ref):
    hh = jax.nn.relu((a0_ref[...] + a1_ref[...]) / dg_ref[...]
                     + b_ref[...] + q_ref[...])
    p_ref[...] = jnp.dot(hh, wl_ref[...], precision=_HIGH,
                         preferred_element_type=jnp.float32)
    qn_ref[...] = jnp.dot(hh, wr_ref[...], precision=_HIGH,
                          preferred_element_type=jnp.float32)


def _comb_body(a0_ref, a1_ref, dg_ref, q_ref, b_ref, h_ref):
    h_ref[...] = jax.nn.relu((a0_ref[...] + a1_ref[...]) / dg_ref[...]
                             + b_ref[...] + q_ref[...])


def _row_spec():
    return pl.BlockSpec((BN, H), lambda i: (i, 0))


def _w_spec():
    return pl.BlockSpec((H, H), lambda i: (0, 0))


def _b_spec():
    return pl.BlockSpec((1, H), lambda i: (0, 0))


def _f32(shape):
    return jax.ShapeDtypeStruct(shape, jnp.float32)


def kernel(x, edge_index, W_emb, b_emb, Wl0, bl0, Wr0, Wl1, bl1, Wr1,
           Wl2, bl2, Wr2, Wl3, bl3, Wr3):
    grid = (N // BN,)
    src3 = edge_index[0].reshape(NW, NCHUNK, C)
    dst3 = edge_index[1].reshape(NW, NCHUNK, C)
    zeros = jnp.zeros((NP, H), jnp.float32)
    zeros16 = jnp.zeros((NP, DW), jnp.float32)

    dacc = _deg_sc(dst3, zeros)
    degc = pl.pallas_call(
        _deg_body, grid=grid,
        in_specs=[pl.BlockSpec((BN, H), lambda i: (i, 0)),
                  pl.BlockSpec((BN, H), lambda i: (i, 0))],
        out_specs=_row_spec(), out_shape=_f32((N, H)),
    )(dacc[:N], dacc[NP:NP + N])

    p, q = pl.pallas_call(
        _emb_pq_body, grid=grid,
        in_specs=[_row_spec(), _w_spec(), _b_spec(), _w_spec(), _w_spec()],
        out_specs=[_row_spec(), _row_spec()],
        out_shape=[_f32((N, H)), _f32((N, H))],
    )(x, W_emb, b_emb.reshape(1, H), Wl0, Wr0)
    p, dacc = lax.optimization_barrier((p, dacc))

    layer_b = [bl0, bl1, bl2, bl3]
    next_w = [(Wl1, Wr1), (Wl2, Wr2), (Wl3, Wr3), None]
    h = None
    for li in range(4):
        acc = _segsum_sc(p, src3, dst3, zeros)
        a0, a1 = acc[:N], acc[NP:NP + N]
        bl = layer_b[li].reshape(1, H)
        if next_w[li] is not None:
            wl_n, wr_n = next_w[li]
            p, q = pl.pallas_call(
                _comb_pq_body, grid=grid,
                in_specs=[_row_spec(), _row_spec(), _row_spec(), _row_spec(),
                          _b_spec(), _w_spec(), _w_spec()],
                out_specs=[_row_spec(), _row_spec()],
                out_shape=[_f32((N, H)), _f32((N, H))],
            )(a0, a1, degc, q, bl, wl_n, wr_n)
        else:
            h = pl.pallas_call(
                _comb_body, grid=grid,
                in_specs=[_row_spec(), _row_spec(), _row_spec(), _row_spec(),
                          _b_spec()],
                out_specs=_row_spec(), out_shape=_f32((N, H)),
            )(a0, a1, degc, q, bl)
    return h


# speedup vs baseline: 1.4763x; 1.0000x over previous
import functools

import jax
import jax.numpy as jnp
from jax import lax
from jax.experimental import pallas as pl
from jax.experimental.pallas import tpu as pltpu
from jax.experimental.pallas import tpu_sc as plsc

N = 10000
NP = 10240
E = 320000
H = 128
NC = 2
NS = 16
NW = NC * NS
EPW = E // NW
C = 80
NCHUNK = 125
RPT = NP // NS

_HIGH = lax.Precision.HIGHEST

_mesh = plsc.VectorSubcoreMesh(
    core_axis_name="c", subcore_axis_name="s", num_cores=NC, num_subcores=NS)


@functools.partial(
    pl.kernel,
    mesh=_mesh,
    out_type=jax.ShapeDtypeStruct((2 * NP, H), jnp.float32),
    scratch_types=[
        pltpu.VMEM((NCHUNK, C), jnp.int32),
        pltpu.VMEM((NCHUNK, C), jnp.int32),
        pltpu.VMEM((C, H), jnp.float32),
        pltpu.VMEM_SHARED((NP, H), jnp.float32),
        pltpu.SemaphoreType.DMA,
    ],
)
def _segsum_sc(p_hbm, src3, dst3, zeros_hbm, out_hbm,
               sslab, dslab, rows, acc, sem):
    cid = lax.axis_index("c")
    sid = lax.axis_index("s")
    wid = cid * NS + sid
    pltpu.sync_copy(zeros_hbm.at[pl.ds(sid * RPT, RPT), :],
                    acc.at[pl.ds(sid * RPT, RPT), :])
    pltpu.sync_copy(src3.at[wid], sslab)
    pltpu.sync_copy(dst3.at[wid], dslab)
    plsc.subcore_barrier()

    def body(i, carry):
        pltpu.async_copy(p_hbm.at[sslab.at[i]], rows, sem).wait()
        pltpu.sync_copy(rows, acc.at[dslab.at[i]], add=True)
        return carry

    lax.fori_loop(0, NCHUNK, body, 0)
    plsc.subcore_barrier()
    pltpu.sync_copy(acc.at[pl.ds(sid * RPT, RPT), :],
                    out_hbm.at[pl.ds(cid * NP + sid * RPT, RPT), :])


@functools.partial(
    pl.kernel,
    mesh=_mesh,
    out_type=jax.ShapeDtypeStruct((2 * NP, H), jnp.float32),
    scratch_types=[
        pltpu.VMEM((NCHUNK, C), jnp.int32),
        pltpu.VMEM((C, H), jnp.float32),
        pltpu.VMEM_SHARED((NP, H), jnp.float32),
    ],
)
def _deg_sc(dst3, zeros_hbm, out_hbm, dslab, ones_v, deg_sh):
    cid = lax.axis_index("c")
    sid = lax.axis_index("s")
    wid = cid * NS + sid
    pltpu.sync_copy(zeros_hbm.at[pl.ds(sid * RPT, RPT), :],
                    deg_sh.at[pl.ds(sid * RPT, RPT), :])
    pltpu.sync_copy(dst3.at[wid], dslab)

    def fill(j, carry):
        for l in range(H // 16):
            ones_v[j, pl.ds(l * 16, 16)] = jnp.full((16,), 1.0, jnp.float32)
        return carry

    lax.fori_loop(0, C, fill, 0)
    plsc.subcore_barrier()

    def body(k, carry):
        pltpu.sync_copy(ones_v, deg_sh.at[dslab.at[k]], add=True)
        return carry

    lax.fori_loop(0, NCHUNK, body, 0)
    plsc.subcore_barrier()
    pltpu.sync_copy(deg_sh.at[pl.ds(sid * RPT, RPT), :],
                    out_hbm.at[pl.ds(cid * NP + sid * RPT, RPT), :])


BN = 2000


def _emb_pq_body(x_ref, we_ref, be_ref, wl_ref, wr_ref, p_ref, q_ref):
    hh = jax.nn.relu(
        jnp.dot(x_ref[...], we_ref[...], precision=_HIGH,
                preferred_element_type=jnp.float32) + be_ref[...])
    p_ref[...] = jnp.dot(hh, wl_ref[...], precision=_HIGH,
                         preferred_element_type=jnp.float32)
    q_ref[...] = jnp.dot(hh, wr_ref[...], precision=_HIGH,
                         preferred_element_type=jnp.float32)


def _deg_body(d0_ref, d1_ref, out_ref):
    d = jnp.maximum(d0_ref[:, 0:1] + d1_ref[:, 0:1], 1.0)
    out_ref[...] = jnp.broadcast_to(d, (BN, H))


def _comb_pq_body(a0_ref, a1_ref, dg_ref, q_ref, b_ref, wl_ref, wr_ref,
                  p_ref, qn_ref):
    hh = jax.nn.relu((a0_ref[...] + a1_ref[...]) / dg_ref[...]
                     + b_ref[...] + q_ref[...])
    p_ref[...] = jnp.dot(hh, wl_ref[...], precision=_HIGH,
                         preferred_element_type=jnp.float32)
    qn_ref[...] = jnp.dot(hh, wr_ref[...], precision=_HIGH,
                          preferred_element_type=jnp.float32)


def _comb_body(a0_ref, a1_ref, dg_ref, q_ref, b_ref, h_ref):
    h_ref[...] = jax.nn.relu((a0_ref[...] + a1_ref[...]) / dg_ref[...]
                             + b_ref[...] + q_ref[...])


def _row_spec():
    return pl.BlockSpec((BN, H), lambda i: (i, 0))


def _w_spec():
    return pl.BlockSpec((H, H), lambda i: (0, 0))


def _b_spec():
    return pl.BlockSpec((1, H), lambda i: (0, 0))


def _f32(shape):
    return jax.ShapeDtypeStruct(shape, jnp.float32)


def kernel(x, edge_index, W_emb, b_emb, Wl0, bl0, Wr0, Wl1, bl1, Wr1,
           Wl2, bl2, Wr2, Wl3, bl3, Wr3):
    grid = (N // BN,)
    src3 = edge_index[0].reshape(NW, NCHUNK, C)
    dst3 = edge_index[1].reshape(NW, NCHUNK, C)
    zeros = jnp.zeros((NP, H), jnp.float32)

    dacc = _deg_sc(dst3, zeros)
    degc = pl.pallas_call(
        _deg_body, grid=grid,
        in_specs=[pl.BlockSpec((BN, H), lambda i: (i, 0)),
                  pl.BlockSpec((BN, H), lambda i: (i, 0))],
        out_specs=_row_spec(), out_shape=_f32((N, H)),
    )(dacc[:N], dacc[NP:NP + N])

    p, q = pl.pallas_call(
        _emb_pq_body, grid=grid,
        in_specs=[_row_spec(), _w_spec(), _b_spec(), _w_spec(), _w_spec()],
        out_specs=[_row_spec(), _row_spec()],
        out_shape=[_f32((N, H)), _f32((N, H))],
    )(x, W_emb, b_emb.reshape(1, H), Wl0, Wr0)
    p, dacc = lax.optimization_barrier((p, dacc))

    layer_b = [bl0, bl1, bl2, bl3]
    next_w = [(Wl1, Wr1), (Wl2, Wr2), (Wl3, Wr3), None]
    h = None
    for li in range(4):
        acc = _segsum_sc(p, src3, dst3, zeros)
        a0, a1 = acc[:N], acc[NP:NP + N]
        bl = layer_b[li].reshape(1, H)
        if next_w[li] is not None:
            wl_n, wr_n = next_w[li]
            p, q = pl.pallas_call(
                _comb_pq_body, grid=grid,
                in_specs=[_row_spec(), _row_spec(), _row_spec(), _row_spec(),
                          _b_spec(), _w_spec(), _w_spec()],
                out_specs=[_row_spec(), _row_spec()],
                out_shape=[_f32((N, H)), _f32((N, H))],
            )(a0, a1, degc, q, bl, wl_n, wr_n)
        else:
            h = pl.pallas_call(
                _comb_body, grid=grid,
                in_specs=[_row_spec(), _row_spec(), _row_spec(), _row_spec(),
                          _b_spec()],
                out_specs=_row_spec(), out_shape=_f32((N, H)),
            )(a0, a1, degc, q, bl)
    return h
